# baseline (speedup 1.0000x reference)
import functools

import jax
import jax.numpy as jnp
from jax import lax
from jax.experimental import pallas as pl
from jax.experimental.pallas import tpu as pltpu
from jax.experimental.pallas import tpu_sc as plsc

H = 1024
F = 3584
E = 8
TOPK = 2
T = 2048
S = T * TOPK
BT = 128
NB = S // BT + E
FT = 512
NF = F // FT

NC = 2
NS = 16
NW = NC * NS

_SC_MESH = plsc.VectorSubcoreMesh(core_axis_name="c", subcore_axis_name="s")


TB = 256


def _router_body(x_ref, gw_ref, logits_ref, idx_ref, w_ref):
    x = x_ref[...]
    gw = gw_ref[...]
    logits = lax.dot_general(x, gw, (((1,), (1,)), ((), ())),
                             preferred_element_type=jnp.float32)
    logits_ref[...] = logits
    lane = lax.broadcasted_iota(jnp.int32, (TB, E), 1)
    m1 = jnp.max(logits, axis=1, keepdims=True)
    i1 = jnp.min(jnp.where(logits == m1, lane, E), axis=1, keepdims=True)
    masked = jnp.where(lane == i1, -jnp.float32(1e30), logits)
    m2 = jnp.max(masked, axis=1, keepdims=True)
    i2 = jnp.min(jnp.where(masked == m2, lane, E), axis=1, keepdims=True)
    p1 = 1.0 / (1.0 + jnp.exp(m2 - m1))
    idx_ref[...] = jnp.concatenate([i1, i2], axis=1)
    w_ref[...] = jnp.concatenate([p1, 1.0 - p1], axis=1)


def _router(x, gate_w):
    return pl.pallas_call(
        _router_body,
        grid=(T // TB,),
        in_specs=[
            pl.BlockSpec((TB, H), lambda i: (i, 0)),
            pl.BlockSpec((E, H), lambda i: (0, 0)),
        ],
        out_specs=[
            pl.BlockSpec((TB, E), lambda i: (i, 0)),
            pl.BlockSpec((TB, TOPK), lambda i: (i, 0)),
            pl.BlockSpec((TB, TOPK), lambda i: (i, 0)),
        ],
        out_shape=[
            jax.ShapeDtypeStruct((T, E), jnp.float32),
            jax.ShapeDtypeStruct((T, TOPK), jnp.int32),
            jax.ShapeDtypeStruct((T, TOPK), jnp.float32),
        ],
    )(x, gate_w)


RPW = NB * BT // NW
DCH = 32


@functools.partial(
    pl.kernel,
    mesh=_SC_MESH,
    out_type=jax.ShapeDtypeStruct((NB * BT, H), jnp.float32),
    scratch_types=[
        pltpu.VMEM((DCH,), jnp.int32),
        pltpu.VMEM((DCH, H), jnp.float32),
        pltpu.SemaphoreType.DMA,
    ],
)
def _dispatch(x_hbm, idx_hbm, xs_hbm, idx_v, rows_v, sem):
    wid = lax.axis_index("s") * NC + lax.axis_index("c")
    base = wid * RPW

    def chunk(c, carry):
        off = base + c * DCH
        pltpu.sync_copy(idx_hbm.at[pl.ds(off, DCH)], idx_v)
        pltpu.async_copy(x_hbm.at[idx_v], rows_v, sem).wait()
        pltpu.sync_copy(rows_v, xs_hbm.at[pl.ds(off, DCH)])
        return carry

    lax.fori_loop(0, RPW // DCH, chunk, 0)


def _moe_body(seid_ref, sval_ref, xs_ref, ws_ref, w1_ref, w3_ref, w2_ref,
              out_ref):
    f = pl.program_id(1)

    @pl.when(sval_ref[pl.program_id(0)] > 0)
    def _():
        x = xs_ref[...]
        g = lax.dot_general(x, w1_ref[0], (((1,), (1,)), ((), ())),
                            preferred_element_type=jnp.float32)
        u = lax.dot_general(x, w3_ref[0], (((1,), (1,)), ((), ())),
                            preferred_element_type=jnp.float32)
        h = (g * jax.nn.sigmoid(g)) * (u * ws_ref[...])
        part = lax.dot_general(h, w2_ref[0], (((1,), (1,)), ((), ())),
                               preferred_element_type=jnp.float32)

        @pl.when(f == 0)
        def _init():
            out_ref[...] = part

        @pl.when(f > 0)
        def _acc():
            out_ref[...] += part


def _grouped_mm(beid, bval, xs, ws, w1, w3, w2):
    grid_spec = pltpu.PrefetchScalarGridSpec(
        num_scalar_prefetch=2,
        grid=(NB, NF),
        in_specs=[
            pl.BlockSpec((BT, H), lambda b, f, seid, sval: (b, 0)),
            pl.BlockSpec((BT, 1), lambda b, f, seid, sval: (b, 0)),
            pl.BlockSpec((1, FT, H), lambda b, f, seid, sval: (seid[b], f, 0)),
            pl.BlockSpec((1, FT, H), lambda b, f, seid, sval: (seid[b], f, 0)),
            pl.BlockSpec((1, H, FT), lambda b, f, seid, sval: (seid[b], 0, f)),
        ],
        out_specs=pl.BlockSpec((BT, H), lambda b, f, seid, sval: (b, 0)),
    )
    return pl.pallas_call(
        _moe_body,
        grid_spec=grid_spec,
        out_shape=jax.ShapeDtypeStruct((NB * BT, H), jnp.float32),
        compiler_params=pltpu.CompilerParams(
            dimension_semantics=("arbitrary", "arbitrary")),
    )(beid, bval, xs, ws, w1, w3, w2)


TPW = T // NW
CCH = 32


@functools.partial(
    pl.kernel,
    mesh=_SC_MESH,
    out_type=jax.ShapeDtypeStruct((T, H), jnp.float32),
    scratch_types=[
        pltpu.VMEM((CCH,), jnp.int32),
        pltpu.VMEM((CCH,), jnp.int32),
        pltpu.VMEM((CCH, H), jnp.float32),
        pltpu.VMEM((CCH, H), jnp.float32),
        pltpu.SemaphoreType.DMA,
        pltpu.SemaphoreType.DMA,
    ],
)
def _combine(y_hbm, pos0_hbm, pos1_hbm, out_hbm, i0_v, i1_v, b0_v, b1_v,
             sem0, sem1):
    wid = lax.axis_index("s") * NC + lax.axis_index("c")
    base = wid * TPW

    def chunk(c, carry):
        off = base + c * CCH
        pltpu.sync_copy(pos0_hbm.at[pl.ds(off, CCH)], i0_v)
        pltpu.sync_copy(pos1_hbm.at[pl.ds(off, CCH)], i1_v)
        cp0 = pltpu.async_copy(y_hbm.at[i0_v], b0_v, sem0)
        cp1 = pltpu.async_copy(y_hbm.at[i1_v], b1_v, sem1)
        cp0.wait()
        cp1.wait()

        def row(r, rc):
            def lanes(j, jc):
                sl = pl.ds(j * 16, 16)
                b0_v[r, sl] = b0_v[r, sl] + b1_v[r, sl]
                return jc
            return lax.fori_loop(0, H // 16, lanes, rc)

        lax.fori_loop(0, CCH, row, 0)
        pltpu.sync_copy(b0_v, out_hbm.at[pl.ds(off, CCH)])
        return carry

    lax.fori_loop(0, TPW // CCH, chunk, 0)


def kernel(hidden_states, gate_w, w1, w3, w2):
    orig_shape = hidden_states.shape
    x = hidden_states.reshape(T, H)

    logits, top_idx, top_w = _router(x, gate_w)

    eid = top_idx.reshape(S)
    wslot = top_w.reshape(S)
    tok = jnp.arange(S, dtype=jnp.int32) // TOPK
    onehot = (eid[:, None] == jnp.arange(E, dtype=jnp.int32)[None, :])
    onehot = onehot.astype(jnp.int32)
    ranks = jnp.cumsum(onehot, axis=0) - onehot
    rank = jnp.take_along_axis(ranks, eid[:, None], axis=1)[:, 0]
    counts = jnp.sum(onehot, axis=0)
    padded = ((counts + BT - 1) // BT) * BT
    astart = jnp.concatenate(
        [jnp.zeros((1,), jnp.int32), jnp.cumsum(padded)[:-1]])
    pos = astart[eid] + rank
    tok_src = jnp.zeros((NB * BT,), jnp.int32).at[pos].set(tok)
    ws_arr = jnp.zeros((NB * BT,), jnp.float32).at[pos].set(wslot)
    ws_arr = ws_arr.reshape(NB * BT, 1)
    bstart = jnp.arange(NB, dtype=jnp.int32) * BT
    gend = astart + padded
    beid = jnp.minimum(
        jnp.sum((bstart[:, None] >= gend[None, :]).astype(jnp.int32), axis=1),
        E - 1)
    bval = (bstart < (astart + counts)[beid]).astype(jnp.int32)

    xs = _dispatch(x, tok_src)
    y = _grouped_mm(beid, bval, xs, ws_arr, w1, w3, w2)
    pos2 = pos.reshape(T, TOPK)
    out = _combine(y, pos2[:, 0], pos2[:, 1])
    return (out.reshape(orig_shape), logits)
